# baseline (speedup 1.0000x reference)
import functools

import jax
import jax.numpy as jnp
from jax import lax
from jax.experimental import pallas as pl
from jax.experimental.pallas import tpu as pltpu
from jax.experimental.pallas import tpu_sc as plsc

N = 10000
E = 160000
H = 256
HH = 128
G = 64

NS = 16
EPW = E // NS
B = 48
PB = 3 * B
SELS = 10240
TRASH = SELS - 16
NPAD = 10240
CHK = NPAD // 2
DUMP = CHK
AGGR = CHK + 8
SPW = CHK // NS
SCB = 400
PADV = EPW | (DUMP << 14)

BS = 1000
BSE = 2000
GRID = N // BS


def _msg_body(h_hbm, e_hbm, src_hbm, dst_hbm, out_hbm,
              srcb_v, sel_v, rh0_v, rh1_v, rh2_v,
              gidx_v, ge0_v, ge1_v, ge2_v, dstw_v, dsc_v,
              agg_sh, hs0, hs1, hs2, es0, es1, es2):
    c = lax.axis_index("c")
    s = lax.axis_index("s")
    w0 = s * EPW
    lanes = lax.iota(jnp.int32, 16)

    pltpu.sync_copy(src_hbm.at[pl.ds(w0, EPW)], srcb_v.at[pl.ds(0, EPW)])
    srcb_v[pl.ds(EPW, 16)] = jnp.zeros((16,), jnp.int32)

    def addoff(t, _):
        sl = pl.ds(t * 16, 16)
        srcb_v[sl] = srcb_v[sl] + c * N
        return 0
    lax.fori_loop(0, (EPW + 16) // 16, addoff, 0)

    def scanchunk(ch, offs):
        pltpu.sync_copy(dst_hbm.at[pl.ds(w0 + ch * SCB, SCB)], dsc_v)

        def sg(k, offs):
            o0, o1 = offs
            d = dsc_v[pl.ds(k * 16, 16)]
            m0 = d < CHK
            cs = plsc.cumsum(m0.astype(jnp.int32))
            k0 = cs[15]
            pos = lanes + (ch * SCB + k * 16)
            v0 = pos | (d << 14)
            v1 = pos | ((d - CHK) << 14)
            idx0 = jnp.where(m0, o0 + cs - 1, TRASH + lanes)
            idx1 = jnp.where(m0, TRASH + lanes, TRASH - 1 - (o1 + lanes - cs))
            plsc.store_scatter(sel_v, [idx0], v0)
            plsc.store_scatter(sel_v, [idx1], v1)
            return (o0 + k0, o1 + (16 - k0))
        return lax.fori_loop(0, SCB // 16, sg, offs)
    n0, n1 = lax.fori_loop(0, EPW // SCB, scanchunk, (0, 0))

    t0 = ((n0 + PB - 1) // PB) * PB

    def fill0(g, _):
        sl = pl.ds(g * 16, 16)
        v = sel_v[sl]
        sel_v[sl] = jnp.where(lanes + g * 16 >= n0, PADV, v)
        return 0
    lax.fori_loop(n0 // 16, t0 // 16, fill0, 0)

    t1 = ((n1 + PB - 1) // PB) * PB

    def fill1(g, _):
        sl = pl.ds(g * 16, 16)
        v = sel_v[sl]
        sel_v[sl] = jnp.where(lanes + g * 16 < TRASH - n1, PADV, v)
        return 0
    lax.fori_loop((TRASH - t1) // 16, (TRASH - n1 + 15) // 16, fill1, 0)

    nbs = (t0 // B, t1 // B)
    rh = (rh0_v, rh1_v, rh2_v)
    ge = (ge0_v, ge1_v, ge2_v)
    hs = (hs0, hs1, hs2)
    es = (es0, es1, es2)

    for p in range(2):
        lo = p * CHK
        nb = nbs[p]

        if p == 0:
            def start(j):
                return j * B
        else:
            def start(j):
                return TRASH - B - j * B

        def zrow(r, _):
            for f in range(HH // 16):
                rh0_v[r, pl.ds(f * 16, 16)] = jnp.zeros((16,), jnp.float32)
            return 0
        lax.fori_loop(0, 32, zrow, 0)

        def zcopy(j, _):
            pltpu.sync_copy(rh0_v.at[pl.ds(0, 32)],
                            agg_sh.at[pl.ds(s * SPW + j * 32, 32)])
            return 0
        lax.fori_loop(0, SPW // 32, zcopy, 0)
        plsc.subcore_barrier()

        def build_idx(j, ge_v):
            js = start(j)

            def bg(k, _):
                sl = pl.ds(k * 16, 16)
                v = sel_v[pl.ds(js + k * 16, 16)]
                pos = jnp.minimum(v & 0x3FFF, EPW)
                gidx_v[sl] = plsc.load_gather(srcb_v, [pos])
                ge_v[sl] = (c * E + w0) + jnp.minimum(pos, EPW - 1)
                return 0
            lax.fori_loop(0, B // 16, bg, 0)

        def process(j, q, q1, q2):
            pltpu.make_async_copy(e_hbm.at[ge[q]], rh[q], es[q]).wait()
            pltpu.make_async_copy(h_hbm.at[gidx_v], rh[q1], hs[q1]).wait()
            pltpu.async_copy(e_hbm.at[ge[q1]], rh[q1], es[q1], add=True)
            build_idx(j + 2, ge[q2])
            pltpu.async_copy(h_hbm.at[gidx_v], rh[q2], hs[q2])

            def row(t, _):
                for f in range(HH // 16):
                    sl = pl.ds(f * 16, 16)
                    rh[q][t, sl] = jnp.maximum(rh[q][t, sl], 0.0)
                return 0
            lax.fori_loop(0, B, row, 0)

            js = start(j)

            def dfix(k, _):
                sl = pl.ds(k * 16, 16)
                v = sel_v[pl.ds(js + k * 16, 16)]
                dstw_v[sl] = jnp.minimum(v >> 14, DUMP + 7)
                return 0
            lax.fori_loop(0, B // 16, dfix, 0)

            pltpu.sync_copy(rh[q], agg_sh.at[dstw_v], add=True)

        build_idx(0, ge[0])
        pltpu.async_copy(h_hbm.at[gidx_v], rh[0], hs[0])
        pltpu.make_async_copy(h_hbm.at[gidx_v], rh[0], hs[0]).wait()
        pltpu.async_copy(e_hbm.at[ge[0]], rh[0], es[0], add=True)
        build_idx(1, ge[1])
        pltpu.async_copy(h_hbm.at[gidx_v], rh[1], hs[1])

        def triple(t, _):
            process(3 * t, 0, 1, 2)
            process(3 * t + 1, 1, 2, 0)
            process(3 * t + 2, 2, 0, 1)
            return 0
        lax.fori_loop(0, nb // 3, triple, 0)

        pltpu.make_async_copy(e_hbm.at[ge[0]], rh[0], es[0]).wait()
        pltpu.make_async_copy(h_hbm.at[gidx_v], rh[1], hs[1]).wait()

        plsc.subcore_barrier()
        pltpu.sync_copy(agg_sh.at[pl.ds(s * SPW, SPW)],
                        out_hbm.at[c, pl.ds(lo + s * SPW, SPW)])


_msg_call = functools.partial(
    pl.kernel,
    mesh=plsc.VectorSubcoreMesh(core_axis_name="c", subcore_axis_name="s"),
    out_type=jax.ShapeDtypeStruct((2, NPAD, HH), jnp.float32),
    compiler_params=pltpu.CompilerParams(needs_layout_passes=False),
    scratch_types=[
        pltpu.VMEM((EPW + 16,), jnp.int32),
        pltpu.VMEM((SELS,), jnp.int32),
        pltpu.VMEM((B, HH), jnp.float32),
        pltpu.VMEM((B, HH), jnp.float32),
        pltpu.VMEM((B, HH), jnp.float32),
        pltpu.VMEM((B,), jnp.int32),
        pltpu.VMEM((B,), jnp.int32),
        pltpu.VMEM((B,), jnp.int32),
        pltpu.VMEM((B,), jnp.int32),
        pltpu.VMEM((B,), jnp.int32),
        pltpu.VMEM((SCB,), jnp.int32),
        pltpu.VMEM_SHARED((AGGR, HH), jnp.float32),
        pltpu.SemaphoreType.DMA,
        pltpu.SemaphoreType.DMA,
        pltpu.SemaphoreType.DMA,
        pltpu.SemaphoreType.DMA,
        pltpu.SemaphoreType.DMA,
        pltpu.SemaphoreType.DMA,
    ],
)


def _msg(h_flat, e_flat, src, dst):
    return _msg_call(_msg_body)(h_flat, e_flat, src, dst)


def _enc_body(x_ref, Wn_ref, bn_ref, o_ref):
    h = jnp.dot(x_ref[...], Wn_ref[...],
                preferred_element_type=jnp.float32) + bn_ref[...]
    o_ref[0] = h[:, :HH]
    o_ref[1] = h[:, HH:]


def _eenc_body(a_ref, We_ref, be_ref, o_ref):
    e = jnp.dot(a_ref[...], We_ref[...],
                preferred_element_type=jnp.float32) + be_ref[...]
    o_ref[0] = e[:, :HH]
    o_ref[1] = e[:, HH:]


def _mlp_body(h_ref, a_ref, W1_ref, b1_ref, W2_ref, b2_ref, o_ref):
    z = jnp.concatenate([h_ref[0] + a_ref[0], h_ref[1] + a_ref[1]], axis=1)
    t = jnp.maximum(
        jnp.dot(z, W1_ref[...], preferred_element_type=jnp.float32)
        + b1_ref[...], 0.0)
    y = jnp.maximum(
        jnp.dot(t, W2_ref[...], preferred_element_type=jnp.float32)
        + b2_ref[...], 0.0)
    o_ref[0] = y[:, :HH]
    o_ref[1] = y[:, HH:]


def _pool_body(h_ref, bt_ref, o_ref, c_acc):
    i = pl.program_id(0)

    @pl.when(i == 0)
    def _():
        o_ref[...] = jnp.zeros_like(o_ref)
        c_acc[...] = jnp.zeros_like(c_acc)

    y = jnp.concatenate([h_ref[0], h_ref[1]], axis=1)
    ids = bt_ref[0, 0]
    onehot = (ids[None, :] ==
              lax.broadcasted_iota(jnp.int32, (G, 1), 0)).astype(jnp.float32)
    o_ref[...] += jnp.dot(onehot, y, preferred_element_type=jnp.float32)
    c_acc[...] += jnp.sum(onehot, axis=1, keepdims=True)

    @pl.when(i == GRID - 1)
    def _():
        o_ref[...] = o_ref[...] / jnp.maximum(c_acc[...], 1.0)


def kernel(x, edge_index, edge_attr, batch, Wn, bn, We, be,
           W11, b11, W12, b12, W21, b21, W22, b22):
    src = edge_index[0]
    dst = edge_index[1]

    full = pl.BlockSpec((2, BS, HH), lambda i: (0, i, 0))
    wspec = lambda shp: pl.BlockSpec(shp, lambda i: tuple(0 for _ in shp))

    h0 = pl.pallas_call(
        _enc_body,
        grid=(GRID,),
        in_specs=[pl.BlockSpec((BS, 9), lambda i: (i, 0)),
                  wspec((9, H)), wspec((1, H))],
        out_specs=full,
        out_shape=jax.ShapeDtypeStruct((2, N, HH), jnp.float32),
    )(x, Wn, bn.reshape(1, H))

    e_st = pl.pallas_call(
        _eenc_body,
        grid=(E // BSE,),
        in_specs=[pl.BlockSpec((BSE, 3), lambda i: (i, 0)),
                  wspec((3, H)), wspec((1, H))],
        out_specs=pl.BlockSpec((2, BSE, HH), lambda i: (0, i, 0)),
        out_shape=jax.ShapeDtypeStruct((2, E, HH), jnp.float32),
    )(edge_attr, We, be.reshape(1, H))
    e_flat = e_st.reshape(2 * E, HH)

    mlp_call = pl.pallas_call(
        _mlp_body,
        grid=(GRID,),
        in_specs=[full, full, wspec((H, H)), wspec((1, H)),
                  wspec((H, H)), wspec((1, H))],
        out_specs=full,
        out_shape=jax.ShapeDtypeStruct((2, N, HH), jnp.float32),
    )

    W1s = jnp.stack([W11, W21])
    b1s = jnp.stack([b11.reshape(1, H), b21.reshape(1, H)])
    W2s = jnp.stack([W12, W22])
    b2s = jnp.stack([b12.reshape(1, H), b22.reshape(1, H)])

    def layer(h_st, ws):
        W1, b1, W2, b2 = ws
        agg = _msg(h_st.reshape(2 * N, HH), e_flat, src, dst)
        h_next = mlp_call(h_st, agg, W1, b1, W2, b2)
        return h_next, 0

    h2, _ = lax.scan(layer, h0, (W1s, b1s, W2s, b2s))

    out = pl.pallas_call(
        _pool_body,
        grid=(GRID,),
        in_specs=[full, pl.BlockSpec((1, 1, BS), lambda i: (i, 0, 0))],
        out_specs=pl.BlockSpec((G, H), lambda i: (0, 0)),
        out_shape=jax.ShapeDtypeStruct((G, H), jnp.float32),
        scratch_shapes=[pltpu.VMEM((G, 1), jnp.float32)],
    )(h2, batch.reshape(GRID, 1, BS))
    return out


# speedup vs baseline: 1.1774x; 1.1774x over previous
import functools

import jax
import jax.numpy as jnp
from jax import lax
from jax.experimental import pallas as pl
from jax.experimental.pallas import tpu as pltpu
from jax.experimental.pallas import tpu_sc as plsc

N = 10000
E = 160000
H = 256
HH = 128
G = 64

NS = 16
EPW = E // NS
B = 48
PB = 2 * B
SELS = 10192
TRASH = SELS - 16
NPAD = 10240
CHK = NPAD // 2
DUMP = CHK
AGGR = CHK + 8
SPW = CHK // NS
SCB = 400
PADV = EPW | (DUMP << 14)

BS = 1000
BSE = 2000
GRID = N // BS


def _msg_body(h_hbm, e_hbm, src_hbm, dst_hbm, out_hbm,
              srcb_v, sel_v, rh0_v, rh1_v, re0_v, re1_v,
              gidx_v, geidx_v, dstw_v, dsc0_v, dsc1_v,
              agg_sh, hs0, hs1, es0, es1, ds0, ds1):
    c = lax.axis_index("c")
    s = lax.axis_index("s")
    w0 = s * EPW
    lanes = lax.iota(jnp.int32, 16)

    pltpu.sync_copy(src_hbm.at[pl.ds(w0, EPW)], srcb_v.at[pl.ds(0, EPW)])
    srcb_v[pl.ds(EPW, 16)] = jnp.zeros((16,), jnp.int32)

    def addoff(t, _):
        sl = pl.ds(t * 16, 16)
        srcb_v[sl] = srcb_v[sl] + c * N
        return 0
    lax.fori_loop(0, (EPW + 16) // 16, addoff, 0)

    def chunk_groups(ch, dsc_v, offs):
        def sg(k, offs):
            o0, o1 = offs
            d = dsc_v[pl.ds(k * 16, 16)]
            m0 = d < CHK
            cs = plsc.cumsum(m0.astype(jnp.int32))
            k0 = cs[15]
            pos = lanes + (ch * SCB + k * 16)
            v0 = pos | (d << 14)
            v1 = pos | ((d - CHK) << 14)
            idx0 = jnp.where(m0, o0 + cs - 1, TRASH + lanes)
            idx1 = jnp.where(m0, TRASH + lanes, TRASH - 1 - (o1 + lanes - cs))
            plsc.store_scatter(sel_v, [idx0], v0)
            plsc.store_scatter(sel_v, [idx1], v1)
            return (o0 + k0, o1 + (16 - k0))
        return lax.fori_loop(0, SCB // 16, sg, offs)

    def issue_chunk(ch, dsc_v, dsem):
        pltpu.async_copy(dst_hbm.at[pl.ds(w0 + ch * SCB, SCB)], dsc_v, dsem)

    def wait_chunk(ch, dsc_v, dsem):
        pltpu.make_async_copy(dst_hbm.at[pl.ds(w0 + ch * SCB, SCB)], dsc_v,
                              dsem).wait()

    issue_chunk(0, dsc0_v, ds0)

    def scanpair(t, offs):
        ch = 2 * t
        wait_chunk(ch, dsc0_v, ds0)
        issue_chunk(ch + 1, dsc1_v, ds1)
        offs = chunk_groups(ch, dsc0_v, offs)
        wait_chunk(ch + 1, dsc1_v, ds1)
        issue_chunk(jnp.minimum(ch + 2, EPW // SCB - 1), dsc0_v, ds0)
        offs = chunk_groups(ch + 1, dsc1_v, offs)
        return offs
    offs = lax.fori_loop(0, EPW // SCB // 2, scanpair, (0, 0))
    wait_chunk(EPW // SCB - 1, dsc0_v, ds0)
    n0, n1 = chunk_groups(EPW // SCB - 1, dsc0_v, offs)

    t0 = ((n0 + PB - 1) // PB) * PB

    def fill0(g, _):
        sl = pl.ds(g * 16, 16)
        v = sel_v[sl]
        sel_v[sl] = jnp.where(lanes + g * 16 >= n0, PADV, v)
        return 0
    lax.fori_loop(n0 // 16, t0 // 16, fill0, 0)

    t1 = ((n1 + PB - 1) // PB) * PB

    def fill1(g, _):
        sl = pl.ds(g * 16, 16)
        v = sel_v[sl]
        sel_v[sl] = jnp.where(lanes + g * 16 < TRASH - n1, PADV, v)
        return 0
    lax.fori_loop((TRASH - t1) // 16, (TRASH - n1 + 15) // 16, fill1, 0)

    nbs = (t0 // B, t1 // B)

    for p in range(2):
        lo = p * CHK
        nb = nbs[p]

        if p == 0:
            def start(j):
                return j * B
        else:
            def start(j):
                return TRASH - B - j * B

        def zrow(r, _):
            for f in range(HH // 16):
                rh0_v[r, pl.ds(f * 16, 16)] = jnp.zeros((16,), jnp.float32)
            return 0
        lax.fori_loop(0, 32, zrow, 0)

        def zcopy(j, _):
            pltpu.sync_copy(rh0_v.at[pl.ds(0, 32)],
                            agg_sh.at[pl.ds(s * SPW + j * 32, 32)])
            return 0
        lax.fori_loop(0, SPW // 32, zcopy, 0)
        plsc.subcore_barrier()

        def build_idx(j):
            js = start(j)

            def bg(k, _):
                sl = pl.ds(k * 16, 16)
                v = sel_v[pl.ds(js + k * 16, 16)]
                pos = jnp.minimum(v & 0x3FFF, EPW)
                gidx_v[sl] = plsc.load_gather(srcb_v, [pos])
                geidx_v[sl] = (c * E + w0) + jnp.minimum(pos, EPW - 1)
                return 0
            lax.fori_loop(0, B // 16, bg, 0)

        def process(j, rh_v, re_v, hsem, esem, orh_v, ore_v, ohsem, oesem):
            pltpu.make_async_copy(h_hbm.at[gidx_v], rh_v, hsem).wait()
            pltpu.make_async_copy(e_hbm.at[geidx_v], re_v, esem).wait()
            build_idx(j + 1)
            pltpu.async_copy(h_hbm.at[gidx_v], orh_v, ohsem)
            pltpu.async_copy(e_hbm.at[geidx_v], ore_v, oesem)

            def row(t, _):
                for f in range(HH // 16):
                    sl = pl.ds(f * 16, 16)
                    rh_v[t, sl] = jnp.maximum(rh_v[t, sl] + re_v[t, sl], 0.0)
                return 0
            lax.fori_loop(0, B, row, 0)

            js = start(j)

            def dfix(k, _):
                sl = pl.ds(k * 16, 16)
                v = sel_v[pl.ds(js + k * 16, 16)]
                dstw_v[sl] = jnp.minimum(v >> 14, DUMP + 7)
                return 0
            lax.fori_loop(0, B // 16, dfix, 0)

            pltpu.sync_copy(rh_v, agg_sh.at[dstw_v], add=True)

        build_idx(0)
        pltpu.async_copy(h_hbm.at[gidx_v], rh0_v, hs0)
        pltpu.async_copy(e_hbm.at[geidx_v], re0_v, es0)

        def pair(t, _):
            process(2 * t, rh0_v, re0_v, hs0, es0, rh1_v, re1_v, hs1, es1)
            process(2 * t + 1, rh1_v, re1_v, hs1, es1, rh0_v, re0_v, hs0,
                    es0)
            return 0
        lax.fori_loop(0, nb // 2, pair, 0)

        pltpu.make_async_copy(h_hbm.at[gidx_v], rh0_v, hs0).wait()
        pltpu.make_async_copy(e_hbm.at[geidx_v], re0_v, es0).wait()

        plsc.subcore_barrier()
        pltpu.sync_copy(agg_sh.at[pl.ds(s * SPW, SPW)],
                        out_hbm.at[c, pl.ds(lo + s * SPW, SPW)])


_msg_call = functools.partial(
    pl.kernel,
    mesh=plsc.VectorSubcoreMesh(core_axis_name="c", subcore_axis_name="s"),
    out_type=jax.ShapeDtypeStruct((2, NPAD, HH), jnp.float32),
    compiler_params=pltpu.CompilerParams(needs_layout_passes=False),
    scratch_types=[
        pltpu.VMEM((EPW + 16,), jnp.int32),
        pltpu.VMEM((SELS,), jnp.int32),
        pltpu.VMEM((B, HH), jnp.float32),
        pltpu.VMEM((B, HH), jnp.float32),
        pltpu.VMEM((B, HH), jnp.float32),
        pltpu.VMEM((B, HH), jnp.float32),
        pltpu.VMEM((B,), jnp.int32),
        pltpu.VMEM((B,), jnp.int32),
        pltpu.VMEM((B,), jnp.int32),
        pltpu.VMEM((SCB,), jnp.int32),
        pltpu.VMEM((SCB,), jnp.int32),
        pltpu.VMEM_SHARED((AGGR, HH), jnp.float32),
        pltpu.SemaphoreType.DMA,
        pltpu.SemaphoreType.DMA,
        pltpu.SemaphoreType.DMA,
        pltpu.SemaphoreType.DMA,
        pltpu.SemaphoreType.DMA,
        pltpu.SemaphoreType.DMA,
    ],
)


def _msg(h_flat, e_flat, src, dst):
    return _msg_call(_msg_body)(h_flat, e_flat, src, dst)


def _enc_body(x_ref, Wn_ref, bn_ref, o_ref):
    h = jnp.dot(x_ref[...], Wn_ref[...],
                preferred_element_type=jnp.float32) + bn_ref[...]
    o_ref[0] = h[:, :HH]
    o_ref[1] = h[:, HH:]


def _eenc_body(a_ref, We_ref, be_ref, o_ref):
    e = jnp.dot(a_ref[...], We_ref[...],
                preferred_element_type=jnp.float32) + be_ref[...]
    o_ref[0] = e[:, :HH]
    o_ref[1] = e[:, HH:]


def _mlp_body(h_ref, a_ref, W1_ref, b1_ref, W2_ref, b2_ref, o_ref):
    z = jnp.concatenate([h_ref[0] + a_ref[0], h_ref[1] + a_ref[1]], axis=1)
    t = jnp.maximum(
        jnp.dot(z, W1_ref[...], preferred_element_type=jnp.float32)
        + b1_ref[...], 0.0)
    y = jnp.maximum(
        jnp.dot(t, W2_ref[...], preferred_element_type=jnp.float32)
        + b2_ref[...], 0.0)
    o_ref[0] = y[:, :HH]
    o_ref[1] = y[:, HH:]


def _pool_body(h_ref, bt_ref, o_ref, c_acc):
    i = pl.program_id(0)

    @pl.when(i == 0)
    def _():
        o_ref[...] = jnp.zeros_like(o_ref)
        c_acc[...] = jnp.zeros_like(c_acc)

    y = jnp.concatenate([h_ref[0], h_ref[1]], axis=1)
    ids = bt_ref[0, 0]
    onehot = (ids[None, :] ==
              lax.broadcasted_iota(jnp.int32, (G, 1), 0)).astype(jnp.float32)
    o_ref[...] += jnp.dot(onehot, y, preferred_element_type=jnp.float32)
    c_acc[...] += jnp.sum(onehot, axis=1, keepdims=True)

    @pl.when(i == GRID - 1)
    def _():
        o_ref[...] = o_ref[...] / jnp.maximum(c_acc[...], 1.0)


def kernel(x, edge_index, edge_attr, batch, Wn, bn, We, be,
           W11, b11, W12, b12, W21, b21, W22, b22):
    src = edge_index[0]
    dst = edge_index[1]

    full = pl.BlockSpec((2, BS, HH), lambda i: (0, i, 0))
    wspec = lambda shp: pl.BlockSpec(shp, lambda i: tuple(0 for _ in shp))

    h0 = pl.pallas_call(
        _enc_body,
        grid=(GRID,),
        in_specs=[pl.BlockSpec((BS, 9), lambda i: (i, 0)),
                  wspec((9, H)), wspec((1, H))],
        out_specs=full,
        out_shape=jax.ShapeDtypeStruct((2, N, HH), jnp.float32),
    )(x, Wn, bn.reshape(1, H))

    e_st = pl.pallas_call(
        _eenc_body,
        grid=(E // BSE,),
        in_specs=[pl.BlockSpec((BSE, 3), lambda i: (i, 0)),
                  wspec((3, H)), wspec((1, H))],
        out_specs=pl.BlockSpec((2, BSE, HH), lambda i: (0, i, 0)),
        out_shape=jax.ShapeDtypeStruct((2, E, HH), jnp.float32),
    )(edge_attr, We, be.reshape(1, H))
    e_flat = e_st.reshape(2 * E, HH)

    mlp_call = pl.pallas_call(
        _mlp_body,
        grid=(GRID,),
        in_specs=[full, full, wspec((H, H)), wspec((1, H)),
                  wspec((H, H)), wspec((1, H))],
        out_specs=full,
        out_shape=jax.ShapeDtypeStruct((2, N, HH), jnp.float32),
    )

    W1s = jnp.stack([W11, W21])
    b1s = jnp.stack([b11.reshape(1, H), b21.reshape(1, H)])
    W2s = jnp.stack([W12, W22])
    b2s = jnp.stack([b12.reshape(1, H), b22.reshape(1, H)])

    def layer(h_st, ws):
        W1, b1, W2, b2 = ws
        agg = _msg(h_st.reshape(2 * N, HH), e_flat, src, dst)
        h_next = mlp_call(h_st, agg, W1, b1, W2, b2)
        return h_next, 0

    h2, _ = lax.scan(layer, h0, (W1s, b1s, W2s, b2s))

    out = pl.pallas_call(
        _pool_body,
        grid=(GRID,),
        in_specs=[full, pl.BlockSpec((1, 1, BS), lambda i: (i, 0, 0))],
        out_specs=pl.BlockSpec((G, H), lambda i: (0, 0)),
        out_shape=jax.ShapeDtypeStruct((G, H), jnp.float32),
        scratch_shapes=[pltpu.VMEM((G, 1), jnp.float32)],
    )(h2, batch.reshape(GRID, 1, BS))
    return out
